# Initial kernel scaffold; baseline (speedup 1.0000x reference)
import jax
import jax.numpy as jnp
from jax.experimental import pallas as pl


def kernel(x, pos_table):
    raise NotImplementedError("write your pallas kernel here")


# speedup vs baseline: 2.9063x; 2.9063x over previous
import jax
import jax.numpy as jnp
from jax.experimental import pallas as pl

BLK_S = 512


def _add_body(x_ref, pos_ref, out_ref):
    out_ref[...] = x_ref[...] + pos_ref[...][None]


def kernel(x, pos_table):
    B, S, D = x.shape
    n_s = S // BLK_S
    return pl.pallas_call(
        _add_body,
        grid=(n_s, B),
        in_specs=[
            pl.BlockSpec((1, BLK_S, D), lambda i, b: (b, i, 0)),
            pl.BlockSpec((BLK_S, D), lambda i, b: (i, 0)),
        ],
        out_specs=pl.BlockSpec((1, BLK_S, D), lambda i, b: (b, i, 0)),
        out_shape=jax.ShapeDtypeStruct((B, S, D), x.dtype),
    )(x, pos_table[:S])
